# baseline (speedup 1.0000x reference)
import functools

import jax
import jax.numpy as jnp
from jax import lax
from jax.experimental import pallas as pl
from jax.experimental.pallas import tpu as pltpu
from jax.experimental.pallas import tpu_sc as plsc

D = 128

NC, NS, LN = 2, 16, 16
NW = NC * NS
_SC_MESH = plsc.VectorSubcoreMesh(
    core_axis_name="c", subcore_axis_name="s", num_cores=NC, num_subcores=NS)


def _fill_rows(ref, nrows, value, dtype):
    v = jnp.full((LN,), value, dtype)

    def body(r, _):
        for k in range(D // LN):
            ref[r, pl.ds(LN * k, LN)] = v
        return 0

    lax.fori_loop(0, nrows, body, 0)


def _add_rows(dst, src, nrows):
    def body(r, _):
        for k in range(D // LN):
            dst[r, pl.ds(LN * k, LN)] = (dst[r, pl.ds(LN * k, LN)]
                                         + src[r, pl.ds(LN * k, LN)])
        return 0

    lax.fori_loop(0, nrows, body, 0)


@functools.lru_cache(maxsize=None)
def _build_segsum(phases, S_acc, S_out, counts=False):
    assert S_acc % NS == 0 and S_out % NS == 0
    dummy = S_acc - 1
    max_nblk = 0
    for nv, R in phases:
        share = R // NW
        assert R % NW == 0 and share % 8 == 0
        max_nblk = max(max_nblk, (share + 127) // 128)
    multi = any(nv > 1 for nv, _ in phases)

    out_types = [jax.ShapeDtypeStruct((NC, S_out, D), jnp.float32)]
    if counts:
        out_types.append(jax.ShapeDtypeStruct((NC, S_out, D), jnp.float32))

    scratch = [
        pltpu.VMEM_SHARED((S_acc, D), jnp.float32),
        pltpu.VMEM((128, D), jnp.float32),
        pltpu.VMEM((max_nblk, 128), jnp.int32),
        pltpu.VMEM((128, D), jnp.float32),
    ]
    if multi:
        scratch.append(pltpu.VMEM((128, D), jnp.float32))
    if counts:
        scratch.append(pltpu.VMEM_SHARED((S_acc, D), jnp.float32))
        scratch.append(pltpu.VMEM((128, D), jnp.float32))

    def body(*refs):
        n_in = sum(1 + nv for nv, _ in phases)
        ins = refs[:n_in]
        n_out = 2 if counts else 1
        outs = refs[n_in:n_in + n_out]
        scr = list(refs[n_in + n_out:])
        acc = scr.pop(0)
        zbuf = scr.pop(0)
        idx2d = scr.pop(0)
        rows = scr.pop(0)
        rows2 = scr.pop(0) if multi else None
        cacc = scr.pop(0) if counts else None
        ones = scr.pop(0) if counts else None

        cid = lax.axis_index("c")
        sid = lax.axis_index("s")
        w = cid * NS + sid

        _fill_rows(zbuf, 128, 0.0, jnp.float32)
        if counts:
            _fill_rows(ones, 128, 1.0, jnp.float32)

        zpt = S_acc // NS
        zdone = 0
        while zdone < zpt:
            zn = min(128, zpt - zdone)
            pltpu.sync_copy(zbuf.at[pl.ds(0, zn), :],
                            acc.at[pl.ds(sid * zpt + zdone, zn), :])
            if counts:
                pltpu.sync_copy(zbuf.at[pl.ds(0, zn), :],
                                cacc.at[pl.ds(sid * zpt + zdone, zn), :])
            zdone += zn
        plsc.subcore_barrier()

        argp = 0
        for nv, R in phases:
            idx_hbm = ins[argp]
            vals = ins[argp + 1:argp + 1 + nv]
            argp += 1 + nv
            share = R // NW
            base = w * share
            nb = share // 128
            tail = share % 128
            nblk = nb + (1 if tail else 0)
            _fill_rows(idx2d, nblk, dummy, jnp.int32)
            for j in range(nb):
                pltpu.sync_copy(idx_hbm.at[pl.ds(base + 128 * j, 128)],
                                idx2d.at[j])
            if tail:
                pltpu.sync_copy(idx_hbm.at[pl.ds(base + 128 * nb, tail)],
                                idx2d.at[nb, pl.ds(0, tail)])
            for j in range(nblk):
                ln = 128 if j < nb else tail
                pltpu.sync_copy(vals[0].at[pl.ds(base + 128 * j, ln), :],
                                rows.at[pl.ds(0, ln), :])
                for v in vals[1:]:
                    pltpu.sync_copy(v.at[pl.ds(base + 128 * j, ln), :],
                                    rows2.at[pl.ds(0, ln), :])
                    _add_rows(rows, rows2, ln)
                pltpu.sync_copy(rows, acc.at[idx2d.at[j]], add=True)
                if counts:
                    pltpu.sync_copy(ones, cacc.at[idx2d.at[j]], add=True)
        plsc.subcore_barrier()

        spt = S_out // NS
        pltpu.sync_copy(acc.at[pl.ds(sid * spt, spt), :],
                        outs[0].at[cid, pl.ds(sid * spt, spt), :])
        if counts:
            pltpu.sync_copy(cacc.at[pl.ds(sid * spt, spt), :],
                            outs[1].at[cid, pl.ds(sid * spt, spt), :])

    return pl.kernel(body, out_type=tuple(out_types) if counts else out_types[0],
                     mesh=_SC_MESH, scratch_types=scratch,
                     compiler_params=pltpu.CompilerParams(
                         needs_layout_passes=False))


E_REAL = 160000
C_CH = 4096
NCHUNK = 40
E_PAD = C_CH * NCHUNK
ACC_ROWS = C_CH + 128
T_REAL = 200000
T_PAD = 229376
TSH = T_PAD // NS
TSB = 2048
TSBR = TSB // 128
SELR = TSBR + 2
ESH = C_CH // NS


def _mul_rows(dst, src, nrows):
    def body(r, _):
        for k in range(D // LN):
            dst[r, pl.ds(LN * k, LN)] = (dst[r, pl.ds(LN * k, LN)]
                                         * src[r, pl.ds(LN * k, LN)])
        return 0

    lax.fori_loop(0, nrows, body, 0)


def _trimsg_body(ea0, eb0, t0, ea1, eb1, t1, ea2, eb2, t2, ei0, ei1v, h0,
                 out, acc, s0, s1, s2, g0, g1, d0, ra0, rb0, ra1, rb1,
                 ia, ib, edst, semA, semB):
    cid = lax.axis_index("c")
    sid = lax.axis_index("s")
    dummy = C_CH
    dump = (SELR - 1) * 128
    keys = ((ea0, eb0, t0), (ea1, eb1, t1), (ea2, eb2, t2))

    lane = lax.iota(jnp.int32, LN)
    for j in range(ESH // 128):
        for k in range(128 // LN):
            edst[j, pl.ds(LN * k, LN)] = sid * ESH + 128 * j + LN * k + lane

    def mk_mul(ra, rb):
        def mulblk(r, _):
            for u in range(4):
                for k in range(D // LN):
                    ra[4 * r + u, pl.ds(LN * k, LN)] = (
                        ra[4 * r + u, pl.ds(LN * k, LN)]
                        * rb[4 * r + u, pl.ds(LN * k, LN)])
            return 0

        return mulblk

    mul0 = mk_mul(ra0, rb0)
    mul1 = mk_mul(ra1, rb1)

    def chunk_body(ci, _):
        ch = NC * ci + cid
        lo = ch * C_CH
        hi = lo + C_CH

        _fill_rows(ra0, 128, 0.0, jnp.float32)
        zpt = ACC_ROWS // NS
        zdone = 0
        while zdone < zpt:
            zn = min(128, zpt - zdone)
            pltpu.sync_copy(ra0.at[pl.ds(0, zn), :],
                            acc.at[pl.ds(sid * zpt + zdone, zn), :])
            zdone += zn
        plsc.subcore_barrier()

        for (ea, eb, tr) in keys:
            def block_body(b, _):
                trow = sid * (TSH // 128) + TSBR * b
                cp0 = pltpu.async_copy(tr.at[0, pl.ds(trow, TSBR), :], s0,
                                       semA)
                cp1 = pltpu.async_copy(tr.at[1, pl.ds(trow, TSBR), :], s1,
                                       semA)
                cp2 = pltpu.async_copy(tr.at[2, pl.ds(trow, TSBR), :], s2,
                                       semA)
                cp0.wait()
                cp1.wait()
                cp2.wait()

                def scan_body(i, off):
                    r8, c8 = i // 8, (i % 8) * LN
                    t2v = s2[r8, pl.ds(c8, LN)]
                    m = (t2v >= lo) & (t2v < hi)
                    cum = plsc.cumsum(m.astype(jnp.int32))
                    pos = jnp.where(m, off + cum - 1, dump)
                    pr, pc = pos >> 7, pos & 127
                    plsc.store_scatter(d0, [pr, pc], t2v - lo)
                    plsc.store_scatter(g0, [pr, pc], s0[r8, pl.ds(c8, LN)])
                    plsc.store_scatter(g1, [pr, pc], s1[r8, pl.ds(c8, LN)])
                    return off + cum[LN - 1]

                nsel = lax.fori_loop(0, TSB // LN, scan_body, 0)

                zi = jnp.zeros((LN,), jnp.int32)
                dv = jnp.full((LN,), dummy, jnp.int32)
                for k in range(128 // LN):
                    pos = nsel + LN * k + lane
                    pr, pc = pos >> 7, pos & 127
                    plsc.store_scatter(d0, [pr, pc], dv)
                    plsc.store_scatter(g0, [pr, pc], zi)
                    plsc.store_scatter(g1, [pr, pc], zi)

                nblk = (nsel + 127) // 128

                def fire(j, ra, rb):
                    pltpu.async_copy(ea.at[g0.at[j]], ra, semA)
                    pltpu.async_copy(eb.at[g1.at[j]], rb, semB)

                def drain(ra, rb):
                    pltpu.make_async_copy(ea.at[g0.at[0]], ra, semA).wait()
                    pltpu.make_async_copy(eb.at[g1.at[0]], rb, semB).wait()

                @pl.when(nblk > 0)
                def _():
                    fire(0, ra0, rb0)

                def pair_body(p, _):
                    j0 = 2 * p
                    j1 = j0 + 1

                    @pl.when(j1 < nblk)
                    def _():
                        fire(j1, ra1, rb1)

                    drain(ra0, rb0)
                    lax.fori_loop(0, 32, mul0, 0)
                    pltpu.sync_copy(ra0, acc.at[d0.at[j0]], add=True)

                    @pl.when(j1 < nblk)
                    def _():
                        @pl.when(j1 + 1 < nblk)
                        def _():
                            fire(j1 + 1, ra0, rb0)

                        drain(ra1, rb1)
                        lax.fori_loop(0, 32, mul1, 0)
                        pltpu.sync_copy(ra1, acc.at[d0.at[j1]], add=True)

                    return 0

                lax.fori_loop(0, (nblk + 1) // 2, pair_body, 0)
                return 0

            lax.fori_loop(0, TSH // TSB, block_body, 0)

        ebase = lo + sid * ESH
        for j in range(ESH // 128):
            pltpu.sync_copy(ei0.at[pl.ds(ebase + 128 * j, 128)], ia.at[j])
            pltpu.sync_copy(ei1v.at[pl.ds(ebase + 128 * j, 128)], ib.at[j])
        cpA0 = pltpu.async_copy(h0.at[ia.at[0]], ra0, semA)
        cpB0 = pltpu.async_copy(h0.at[ib.at[0]], rb0, semB)
        cpA1 = pltpu.async_copy(h0.at[ia.at[1]], ra1, semA)
        cpB1 = pltpu.async_copy(h0.at[ib.at[1]], rb1, semB)
        cpA0.wait()
        cpB0.wait()
        lax.fori_loop(0, 32, mul0, 0)
        pltpu.sync_copy(ra0, acc.at[edst.at[0]], add=True)
        cpA1.wait()
        cpB1.wait()
        lax.fori_loop(0, 32, mul1, 0)
        pltpu.sync_copy(ra1, acc.at[edst.at[1]], add=True)
        plsc.subcore_barrier()

        pltpu.sync_copy(acc.at[pl.ds(sid * ESH, ESH), :],
                        out.at[pl.ds(lo + sid * ESH, ESH), :])
        plsc.subcore_barrier()
        return 0

    lax.fori_loop(0, NCHUNK // NC, chunk_body, 0)


@functools.lru_cache(maxsize=None)
def _build_trimsg():
    scratch = [
        pltpu.VMEM_SHARED((ACC_ROWS, D), jnp.float32),
        pltpu.VMEM((TSBR, 128), jnp.int32),
        pltpu.VMEM((TSBR, 128), jnp.int32),
        pltpu.VMEM((TSBR, 128), jnp.int32),
        pltpu.VMEM((SELR, 128), jnp.int32),
        pltpu.VMEM((SELR, 128), jnp.int32),
        pltpu.VMEM((SELR, 128), jnp.int32),
        pltpu.VMEM((128, D), jnp.float32),
        pltpu.VMEM((128, D), jnp.float32),
        pltpu.VMEM((128, D), jnp.float32),
        pltpu.VMEM((128, D), jnp.float32),
        pltpu.VMEM((ESH // 128, 128), jnp.int32),
        pltpu.VMEM((ESH // 128, 128), jnp.int32),
        pltpu.VMEM((ESH // 128, 128), jnp.int32),
        pltpu.SemaphoreType.DMA,
        pltpu.SemaphoreType.DMA,
    ]
    return pl.kernel(
        _trimsg_body,
        out_type=jax.ShapeDtypeStruct((E_PAD, D), jnp.float32),
        mesh=_SC_MESH, scratch_types=scratch,
        compiler_params=pltpu.CompilerParams(needs_layout_passes=False))


def _pad_tri(tri):
    padc = jnp.concatenate(
        [jnp.zeros((2, T_PAD - T_REAL), jnp.int32),
         jnp.full((1, T_PAD - T_REAL), -1, jnp.int32)], axis=0)
    return jnp.concatenate([tri, padc], axis=1).reshape(3, T_PAD // 128, 128)


def _mlp_stats_body(na, *refs):
    addends = refs[:na]
    w1_ref, b1_ref, o_ref = refs[na:]
    i = pl.program_id(0)
    x = addends[0][...]
    for r in addends[1:]:
        x = x + r[...]
    p = jnp.dot(x, w1_ref[...], preferred_element_type=jnp.float32)
    p = p + b1_ref[...]

    @pl.when(i == 0)
    def _():
        o_ref[...] = jnp.zeros_like(o_ref)

    o_ref[0, :] += jnp.sum(p, axis=0)
    o_ref[1, :] += jnp.sum(p * p, axis=0)


def _mlp_apply_body(na, *refs):
    addends = refs[:na]
    w1_ref, b1_ref, st_ref, w2_ref, b2_ref, o_ref = refs[na:]
    x = addends[0][...]
    for r in addends[1:]:
        x = x + r[...]
    p = jnp.dot(x, w1_ref[...], preferred_element_type=jnp.float32)
    p = p + b1_ref[...]
    h = jnp.maximum((p - st_ref[0:1, :]) * st_ref[1:2, :], 0.0)
    o_ref[...] = jnp.dot(h, w2_ref[...], preferred_element_type=jnp.float32) \
        + b2_ref[...]


@functools.partial(jax.jit,
                   static_argnames=("stat_rows", "stat_blk", "out_rows",
                                    "out_blk"))
def _mlp(addends, w1, b1, w2, b2, stat_rows, stat_blk, out_rows, out_blk):
    na = len(addends)
    assert stat_rows % stat_blk == 0 and out_rows % out_blk == 0

    def row_map(i):
        return (i, 0)

    stats = pl.pallas_call(
        functools.partial(_mlp_stats_body, na),
        grid=(stat_rows // stat_blk,),
        in_specs=[pl.BlockSpec((stat_blk, D), row_map)] * na + [
            pl.BlockSpec((D, D), lambda i: (0, 0)),
            pl.BlockSpec((D,), lambda i: (0,)),
        ],
        out_specs=pl.BlockSpec((2, D), lambda i: (0, 0)),
        out_shape=jax.ShapeDtypeStruct((2, D), jnp.float32),
    )(*addends, w1, b1)
    mu = stats[0] * (1.0 / stat_rows)
    var = stats[1] * (1.0 / stat_rows) - mu * mu
    inv_sd = 1.0 / (jnp.sqrt(jnp.maximum(var, 0.0)) + 1e-5)
    st = jnp.stack([mu, inv_sd])

    return pl.pallas_call(
        functools.partial(_mlp_apply_body, na),
        grid=(out_rows // out_blk,),
        in_specs=[pl.BlockSpec((out_blk, D), row_map)] * na + [
            pl.BlockSpec((D, D), lambda i: (0, 0)),
            pl.BlockSpec((D,), lambda i: (0,)),
            pl.BlockSpec((2, D), lambda i: (0, 0)),
            pl.BlockSpec((D, D), lambda i: (0, 0)),
            pl.BlockSpec((D,), lambda i: (0,)),
        ],
        out_specs=pl.BlockSpec((out_blk, D), row_map),
        out_shape=jax.ShapeDtypeStruct((out_rows, D), jnp.float32),
    )(*addends, w1, b1, st, w2, b2)


def _head_body(g_ref, wp1_ref, bp1_ref, wp2_ref, bp2_ref, o_ref):
    h = jnp.dot(g_ref[...], wp1_ref[...], preferred_element_type=jnp.float32)
    h = h + bp1_ref[...]
    h = jnp.where(h > 0, h, jnp.exp(jnp.minimum(h, 0.0)) - 1.0)
    o_ref[...] = jnp.dot(h, wp2_ref[...], preferred_element_type=jnp.float32) \
        + bp2_ref[...]


@jax.jit
def _head(g, wp1, bp1, wp2, bp2):
    return pl.pallas_call(
        _head_body,
        out_shape=jax.ShapeDtypeStruct((g.shape[0], wp2.shape[1]), jnp.float32),
    )(g, wp1, bp1, wp2, bp2)


def _encode(table, idx):
    out = jnp.zeros((idx.shape[0], table.shape[2]), dtype=table.dtype)
    for c in range(table.shape[0]):
        out = out + jnp.take(table[c], idx[:, c], axis=0)
    return out


def kernel(x, edge_attr, edge_index, edge_index2, edge_index3,
           triangle_1_1_1, triangle_1_1_2, triangle_2_2_1, triangle_2_2_2,
           triangle_1_2_3, triangle_3_3_1, triangle_2_2_3, triangle_3_3_2,
           triangle_3_3_3, inverse_edge_1, inverse_edge_2, inverse_edge_3,
           batch0, num_nodes, atom_emb, bond_emb, W1, b1, W2, b2,
           Wp1, bp1, Wp2, bp2):
    nN = x.shape[0]
    nE = edge_attr.shape[0]
    G = 256
    epad = E_PAD - nE

    ei1 = jnp.pad(edge_index, ((0, 0), (0, epad)))
    ei2 = jnp.pad(edge_index2, ((0, 0), (0, epad)))
    ei3 = jnp.pad(edge_index3, ((0, 0), (0, epad)))
    inv1 = jnp.pad(inverse_edge_1, (0, epad))
    inv2 = jnp.pad(inverse_edge_2, (0, epad))
    inv3 = jnp.pad(inverse_edge_3, (0, epad))
    t111 = _pad_tri(triangle_1_1_1)
    t112 = _pad_tri(triangle_1_1_2)
    t221 = _pad_tri(triangle_2_2_1)
    t222 = _pad_tri(triangle_2_2_2)
    t123 = _pad_tri(triangle_1_2_3)
    t331 = _pad_tri(triangle_3_3_1)
    t223 = _pad_tri(triangle_2_2_3)
    t332 = _pad_tri(triangle_3_3_2)
    t333 = _pad_tri(triangle_3_3_3)

    h_atom = _encode(atom_emb, x)
    h_atom = h_atom + jnp.asarray(num_nodes - nN, dtype=h_atom.dtype)
    h0 = h_atom
    e1 = _encode(bond_emb, jnp.pad(edge_attr, ((0, epad), (0, 0))))
    e2 = h_atom[ei2[0]] + h_atom[ei2[1]]
    e3 = h_atom[ei3[0]] + h_atom[ei3[1]]

    segsum_e2n = _build_segsum(((1, nE),), 10240, 10240)
    segsum_node = _build_segsum(((1, nE), (1, nE), (1, nE)), 10240, 10240)
    segsum_pool = _build_segsum(((3, 10240),), 288, 256, counts=True)
    trimsg = _build_trimsg()

    for l in range(W1.shape[0]):
        m0p = segsum_e2n(ei1[1, :nE], e1)
        m1 = trimsg(e1, e1, t111, e2, e2, t221, e3, e3, t331,
                    ei1[0], ei1[1], h0)
        m2 = trimsg(e1, e1, t112, e2, e2, t222, e3, e3, t332,
                    ei2[0], ei2[1], h0)
        m3 = trimsg(e1, e2, t123, e2, e2, t223, e3, e3, t333,
                    ei3[0], ei3[1], h0)
        h0 = _mlp((h0, m0p[0, :nN], m0p[1, :nN]),
                  W1[l, 0], b1[l, 0], W2[l, 0], b2[l, 0],
                  nN, 1000, nN, 1000)
        e1n = _mlp((e1, m1), W1[l, 1], b1[l, 1], W2[l, 1], b2[l, 1],
                   nE, 1280, E_PAD, 1024)
        e1 = 0.5 * (e1n + e1n[inv1])
        e2n = _mlp((e2, m2), W1[l, 2], b1[l, 2], W2[l, 2], b2[l, 2],
                   nE, 1280, E_PAD, 1024)
        e2 = 0.5 * (e2n + e2n[inv2])
        e3n = _mlp((e3, m3), W1[l, 3], b1[l, 3], W2[l, 3], b2[l, 3],
                   nE, 1280, E_PAD, 1024)
        e3 = 0.5 * (e3n + e3n[inv3])

    nsp = segsum_node(ei1[0, :nE], e1, ei2[0, :nE], e2, ei3[0, :nE], e3)
    pad = ((0, 10240 - nN), (0, 0))
    b0p = jnp.pad(batch0, (0, 10240 - nN), constant_values=287)
    gp, cp = segsum_pool(b0p, jnp.pad(h0, pad), jnp.pad(nsp[0, :nN], pad),
                         jnp.pad(nsp[1, :nN], pad))
    cnt = (cp[0] + cp[1])[:, 0:1]
    g = (gp[0] + gp[1]) / jnp.clip(cnt, 1.0)
    return _head(g, Wp1, bp1, Wp2, bp2)


# speedup vs baseline: 17.6849x; 17.6849x over previous
import functools

import jax
import jax.numpy as jnp
from jax import lax
from jax.experimental import pallas as pl
from jax.experimental.pallas import tpu as pltpu
from jax.experimental.pallas import tpu_sc as plsc

D = 128

NC, NS, LN = 2, 16, 16
NW = NC * NS
_SC_MESH = plsc.VectorSubcoreMesh(
    core_axis_name="c", subcore_axis_name="s", num_cores=NC, num_subcores=NS)


def _fill_rows(ref, nrows, value, dtype):
    v = jnp.full((LN,), value, dtype)

    def body(r, _):
        for k in range(D // LN):
            ref[r, pl.ds(LN * k, LN)] = v
        return 0

    lax.fori_loop(0, nrows, body, 0)


def _add_rows(dst, src, nrows):
    def body(r, _):
        for k in range(D // LN):
            dst[r, pl.ds(LN * k, LN)] = (dst[r, pl.ds(LN * k, LN)]
                                         + src[r, pl.ds(LN * k, LN)])
        return 0

    lax.fori_loop(0, nrows, body, 0)


@functools.lru_cache(maxsize=None)
def _build_segsum(phases, S_acc, S_out, counts=False):
    assert S_acc % NS == 0 and S_out % NS == 0
    dummy = S_acc - 1
    max_nblk = 0
    for nv, R in phases:
        share = R // NW
        assert R % NW == 0 and share % 8 == 0
        max_nblk = max(max_nblk, (share + 127) // 128)
    multi = any(nv > 1 for nv, _ in phases)

    out_types = [jax.ShapeDtypeStruct((NC, S_out, D), jnp.float32)]
    if counts:
        out_types.append(jax.ShapeDtypeStruct((NC, S_out, D), jnp.float32))

    scratch = [
        pltpu.VMEM_SHARED((S_acc, D), jnp.float32),
        pltpu.VMEM((128, D), jnp.float32),
        pltpu.VMEM((max_nblk, 128), jnp.int32),
        pltpu.VMEM((128, D), jnp.float32),
    ]
    if multi:
        scratch.append(pltpu.VMEM((128, D), jnp.float32))
    if counts:
        scratch.append(pltpu.VMEM_SHARED((S_acc, D), jnp.float32))
        scratch.append(pltpu.VMEM((128, D), jnp.float32))

    def body(*refs):
        n_in = sum(1 + nv for nv, _ in phases)
        ins = refs[:n_in]
        n_out = 2 if counts else 1
        outs = refs[n_in:n_in + n_out]
        scr = list(refs[n_in + n_out:])
        acc = scr.pop(0)
        zbuf = scr.pop(0)
        idx2d = scr.pop(0)
        rows = scr.pop(0)
        rows2 = scr.pop(0) if multi else None
        cacc = scr.pop(0) if counts else None
        ones = scr.pop(0) if counts else None

        cid = lax.axis_index("c")
        sid = lax.axis_index("s")
        w = cid * NS + sid

        _fill_rows(zbuf, 128, 0.0, jnp.float32)
        if counts:
            _fill_rows(ones, 128, 1.0, jnp.float32)

        zpt = S_acc // NS
        zdone = 0
        while zdone < zpt:
            zn = min(128, zpt - zdone)
            pltpu.sync_copy(zbuf.at[pl.ds(0, zn), :],
                            acc.at[pl.ds(sid * zpt + zdone, zn), :])
            if counts:
                pltpu.sync_copy(zbuf.at[pl.ds(0, zn), :],
                                cacc.at[pl.ds(sid * zpt + zdone, zn), :])
            zdone += zn
        plsc.subcore_barrier()

        argp = 0
        for nv, R in phases:
            idx_hbm = ins[argp]
            vals = ins[argp + 1:argp + 1 + nv]
            argp += 1 + nv
            share = R // NW
            base = w * share
            nb = share // 128
            tail = share % 128
            nblk = nb + (1 if tail else 0)
            _fill_rows(idx2d, nblk, dummy, jnp.int32)
            for j in range(nb):
                pltpu.sync_copy(idx_hbm.at[pl.ds(base + 128 * j, 128)],
                                idx2d.at[j])
            if tail:
                pltpu.sync_copy(idx_hbm.at[pl.ds(base + 128 * nb, tail)],
                                idx2d.at[nb, pl.ds(0, tail)])
            for j in range(nblk):
                ln = 128 if j < nb else tail
                pltpu.sync_copy(vals[0].at[pl.ds(base + 128 * j, ln), :],
                                rows.at[pl.ds(0, ln), :])
                for v in vals[1:]:
                    pltpu.sync_copy(v.at[pl.ds(base + 128 * j, ln), :],
                                    rows2.at[pl.ds(0, ln), :])
                    _add_rows(rows, rows2, ln)
                pltpu.sync_copy(rows, acc.at[idx2d.at[j]], add=True)
                if counts:
                    pltpu.sync_copy(ones, cacc.at[idx2d.at[j]], add=True)
        plsc.subcore_barrier()

        spt = S_out // NS
        pltpu.sync_copy(acc.at[pl.ds(sid * spt, spt), :],
                        outs[0].at[cid, pl.ds(sid * spt, spt), :])
        if counts:
            pltpu.sync_copy(cacc.at[pl.ds(sid * spt, spt), :],
                            outs[1].at[cid, pl.ds(sid * spt, spt), :])

    return pl.kernel(body, out_type=tuple(out_types) if counts else out_types[0],
                     mesh=_SC_MESH, scratch_types=scratch,
                     compiler_params=pltpu.CompilerParams(
                         needs_layout_passes=False))


E_REAL = 160000
C_CH = 4096
NCHUNK = 40
E_PAD = C_CH * NCHUNK
ACC_ROWS = C_CH + 128
T_REAL = 200000
T_PAD = 229376
TSH = T_PAD // NS
TSB = 2048
TSBR = TSB // 128
SELR = TSBR + 2
ESH = C_CH // NS


def _mul_rows(dst, src, nrows):
    def body(r, _):
        for k in range(D // LN):
            dst[r, pl.ds(LN * k, LN)] = (dst[r, pl.ds(LN * k, LN)]
                                         * src[r, pl.ds(LN * k, LN)])
        return 0

    lax.fori_loop(0, nrows, body, 0)


def _trimsg_body(ea0, eb0, t0, ea1, eb1, t1, ea2, eb2, t2, ei0, ei1v, h0,
                 out, acc, s0, s1, s2, g0, g1, d0, ra0, rb0, ra1, rb1,
                 ia, ib, edst, semA, semB):
    cid = lax.axis_index("c")
    sid = lax.axis_index("s")
    dummy = C_CH
    dump = (SELR - 1) * 128
    keys = ((ea0, eb0, t0), (ea1, eb1, t1), (ea2, eb2, t2))

    lane = lax.iota(jnp.int32, LN)
    for j in range(ESH // 128):
        for k in range(128 // LN):
            edst[j, pl.ds(LN * k, LN)] = sid * ESH + 128 * j + LN * k + lane

    def mk_mul(ra, rb):
        def mulblk(r, _):
            for u in range(4):
                for k in range(D // LN):
                    ra[4 * r + u, pl.ds(LN * k, LN)] = (
                        ra[4 * r + u, pl.ds(LN * k, LN)]
                        * rb[4 * r + u, pl.ds(LN * k, LN)])
            return 0

        return mulblk

    mul0 = mk_mul(ra0, rb0)
    mul1 = mk_mul(ra1, rb1)

    def chunk_body(ci, _):
        ch = NC * ci + cid
        lo = ch * C_CH
        hi = lo + C_CH

        _fill_rows(ra0, 128, 0.0, jnp.float32)
        zpt = ACC_ROWS // NS
        zdone = 0
        while zdone < zpt:
            zn = min(128, zpt - zdone)
            pltpu.sync_copy(ra0.at[pl.ds(0, zn), :],
                            acc.at[pl.ds(sid * zpt + zdone, zn), :])
            zdone += zn
        plsc.subcore_barrier()

        for (ea, eb, tr) in keys:
            def block_body(b, _):
                trow = sid * (TSH // 128) + TSBR * b
                cp0 = pltpu.async_copy(tr.at[0, pl.ds(trow, TSBR), :], s0,
                                       semA)
                cp1 = pltpu.async_copy(tr.at[1, pl.ds(trow, TSBR), :], s1,
                                       semA)
                cp2 = pltpu.async_copy(tr.at[2, pl.ds(trow, TSBR), :], s2,
                                       semA)
                cp0.wait()
                cp1.wait()
                cp2.wait()

                def scan_body(i, off):
                    r8, c8 = i // 8, (i % 8) * LN
                    t2v = s2[r8, pl.ds(c8, LN)]
                    m = (t2v >= lo) & (t2v < hi)
                    cum = plsc.cumsum(m.astype(jnp.int32))
                    pos = jnp.where(m, off + cum - 1, dump)
                    pr, pc = pos >> 7, pos & 127
                    plsc.store_scatter(d0, [pr, pc], t2v - lo)
                    plsc.store_scatter(g0, [pr, pc], s0[r8, pl.ds(c8, LN)])
                    plsc.store_scatter(g1, [pr, pc], s1[r8, pl.ds(c8, LN)])
                    return off + cum[LN - 1]

                nsel = lax.fori_loop(0, TSB // LN, scan_body, 0)

                zi = jnp.zeros((LN,), jnp.int32)
                dv = jnp.full((LN,), dummy, jnp.int32)
                for k in range(128 // LN):
                    pos = nsel + LN * k + lane
                    pr, pc = pos >> 7, pos & 127
                    plsc.store_scatter(d0, [pr, pc], dv)
                    plsc.store_scatter(g0, [pr, pc], zi)
                    plsc.store_scatter(g1, [pr, pc], zi)

                nblk = (nsel + 127) // 128

                def fire(j, ra, rb):
                    pltpu.async_copy(ea.at[g0.at[j]], ra, semA)
                    pltpu.async_copy(eb.at[g1.at[j]], rb, semB)

                def drain(ra, rb):
                    pltpu.make_async_copy(ea.at[g0.at[0]], ra, semA).wait()
                    pltpu.make_async_copy(eb.at[g1.at[0]], rb, semB).wait()

                @pl.when(nblk > 0)
                def _():
                    fire(0, ra0, rb0)

                def pair_body(p, _):
                    j0 = 2 * p
                    j1 = j0 + 1

                    @pl.when(j1 < nblk)
                    def _():
                        fire(j1, ra1, rb1)

                    drain(ra0, rb0)
                    lax.fori_loop(0, 32, mul0, 0)
                    pltpu.sync_copy(ra0, acc.at[d0.at[j0]], add=True)

                    @pl.when(j1 < nblk)
                    def _():
                        @pl.when(j1 + 1 < nblk)
                        def _():
                            fire(j1 + 1, ra0, rb0)

                        drain(ra1, rb1)
                        lax.fori_loop(0, 32, mul1, 0)
                        pltpu.sync_copy(ra1, acc.at[d0.at[j1]], add=True)

                    return 0

                lax.fori_loop(0, (nblk + 1) // 2, pair_body, 0)
                return 0

            lax.fori_loop(0, TSH // TSB, block_body, 0)

        ebase = lo + sid * ESH
        for j in range(ESH // 128):
            pltpu.sync_copy(ei0.at[pl.ds(ebase + 128 * j, 128)], ia.at[j])
            pltpu.sync_copy(ei1v.at[pl.ds(ebase + 128 * j, 128)], ib.at[j])
        cpA0 = pltpu.async_copy(h0.at[ia.at[0]], ra0, semA)
        cpB0 = pltpu.async_copy(h0.at[ib.at[0]], rb0, semB)
        cpA1 = pltpu.async_copy(h0.at[ia.at[1]], ra1, semA)
        cpB1 = pltpu.async_copy(h0.at[ib.at[1]], rb1, semB)
        cpA0.wait()
        cpB0.wait()
        lax.fori_loop(0, 32, mul0, 0)
        pltpu.sync_copy(ra0, acc.at[edst.at[0]], add=True)
        cpA1.wait()
        cpB1.wait()
        lax.fori_loop(0, 32, mul1, 0)
        pltpu.sync_copy(ra1, acc.at[edst.at[1]], add=True)
        plsc.subcore_barrier()

        pltpu.sync_copy(acc.at[pl.ds(sid * ESH, ESH), :],
                        out.at[pl.ds(lo + sid * ESH, ESH), :])
        plsc.subcore_barrier()
        return 0

    lax.fori_loop(0, NCHUNK // NC, chunk_body, 0)


@functools.lru_cache(maxsize=None)
def _build_trimsg():
    scratch = [
        pltpu.VMEM_SHARED((ACC_ROWS, D), jnp.float32),
        pltpu.VMEM((TSBR, 128), jnp.int32),
        pltpu.VMEM((TSBR, 128), jnp.int32),
        pltpu.VMEM((TSBR, 128), jnp.int32),
        pltpu.VMEM((SELR, 128), jnp.int32),
        pltpu.VMEM((SELR, 128), jnp.int32),
        pltpu.VMEM((SELR, 128), jnp.int32),
        pltpu.VMEM((128, D), jnp.float32),
        pltpu.VMEM((128, D), jnp.float32),
        pltpu.VMEM((128, D), jnp.float32),
        pltpu.VMEM((128, D), jnp.float32),
        pltpu.VMEM((ESH // 128, 128), jnp.int32),
        pltpu.VMEM((ESH // 128, 128), jnp.int32),
        pltpu.VMEM((ESH // 128, 128), jnp.int32),
        pltpu.SemaphoreType.DMA,
        pltpu.SemaphoreType.DMA,
    ]
    return pl.kernel(
        _trimsg_body,
        out_type=jax.ShapeDtypeStruct((E_PAD, D), jnp.float32),
        mesh=_SC_MESH, scratch_types=scratch,
        compiler_params=pltpu.CompilerParams(needs_layout_passes=False))


def _pad_tri(tri):
    padc = jnp.concatenate(
        [jnp.zeros((2, T_PAD - T_REAL), jnp.int32),
         jnp.full((1, T_PAD - T_REAL), -1, jnp.int32)], axis=0)
    return jnp.concatenate([tri, padc], axis=1).reshape(3, T_PAD // 128, 128)


def _mlp_stats_body(na, *refs):
    addends = refs[:na]
    w1_ref, b1_ref, o_ref = refs[na:]
    i = pl.program_id(0)
    x = addends[0][...]
    for r in addends[1:]:
        x = x + r[...]
    p = jnp.dot(x, w1_ref[...], preferred_element_type=jnp.float32)
    p = p + b1_ref[...]

    @pl.when(i == 0)
    def _():
        o_ref[...] = jnp.zeros_like(o_ref)

    o_ref[0, :] += jnp.sum(p, axis=0)
    o_ref[1, :] += jnp.sum(p * p, axis=0)


def _mlp_apply_body(na, *refs):
    addends = refs[:na]
    w1_ref, b1_ref, st_ref, w2_ref, b2_ref, o_ref = refs[na:]
    x = addends[0][...]
    for r in addends[1:]:
        x = x + r[...]
    p = jnp.dot(x, w1_ref[...], preferred_element_type=jnp.float32)
    p = p + b1_ref[...]
    h = jnp.maximum((p - st_ref[0:1, :]) * st_ref[1:2, :], 0.0)
    o_ref[...] = jnp.dot(h, w2_ref[...], preferred_element_type=jnp.float32) \
        + b2_ref[...]


@functools.partial(jax.jit,
                   static_argnames=("stat_rows", "stat_blk", "out_rows",
                                    "out_blk"))
def _mlp(addends, w1, b1, w2, b2, stat_rows, stat_blk, out_rows, out_blk):
    na = len(addends)
    assert stat_rows % stat_blk == 0 and out_rows % out_blk == 0

    def row_map(i):
        return (i, 0)

    stats = pl.pallas_call(
        functools.partial(_mlp_stats_body, na),
        grid=(stat_rows // stat_blk,),
        in_specs=[pl.BlockSpec((stat_blk, D), row_map)] * na + [
            pl.BlockSpec((D, D), lambda i: (0, 0)),
            pl.BlockSpec((D,), lambda i: (0,)),
        ],
        out_specs=pl.BlockSpec((2, D), lambda i: (0, 0)),
        out_shape=jax.ShapeDtypeStruct((2, D), jnp.float32),
    )(*addends, w1, b1)
    mu = stats[0] * (1.0 / stat_rows)
    var = stats[1] * (1.0 / stat_rows) - mu * mu
    inv_sd = 1.0 / (jnp.sqrt(jnp.maximum(var, 0.0)) + 1e-5)
    st = jnp.stack([mu, inv_sd])

    return pl.pallas_call(
        functools.partial(_mlp_apply_body, na),
        grid=(out_rows // out_blk,),
        in_specs=[pl.BlockSpec((out_blk, D), row_map)] * na + [
            pl.BlockSpec((D, D), lambda i: (0, 0)),
            pl.BlockSpec((D,), lambda i: (0,)),
            pl.BlockSpec((2, D), lambda i: (0, 0)),
            pl.BlockSpec((D, D), lambda i: (0, 0)),
            pl.BlockSpec((D,), lambda i: (0,)),
        ],
        out_specs=pl.BlockSpec((out_blk, D), row_map),
        out_shape=jax.ShapeDtypeStruct((out_rows, D), jnp.float32),
    )(*addends, w1, b1, st, w2, b2)


def _head_body(g_ref, wp1_ref, bp1_ref, wp2_ref, bp2_ref, o_ref):
    h = jnp.dot(g_ref[...], wp1_ref[...], preferred_element_type=jnp.float32)
    h = h + bp1_ref[...]
    h = jnp.where(h > 0, h, jnp.exp(jnp.minimum(h, 0.0)) - 1.0)
    o_ref[...] = jnp.dot(h, wp2_ref[...], preferred_element_type=jnp.float32) \
        + bp2_ref[...]


@jax.jit
def _head(g, wp1, bp1, wp2, bp2):
    return pl.pallas_call(
        _head_body,
        out_shape=jax.ShapeDtypeStruct((g.shape[0], wp2.shape[1]), jnp.float32),
    )(g, wp1, bp1, wp2, bp2)


def _encode(table, idx):
    out = jnp.zeros((idx.shape[0], table.shape[2]), dtype=table.dtype)
    for c in range(table.shape[0]):
        out = out + jnp.take(table[c], idx[:, c], axis=0)
    return out


def _tri_msg_jnp(ea, eb, tri, nseg):
    m = ea[tri[0]] * eb[tri[1]]
    return jax.ops.segment_sum(m, tri[2], num_segments=nseg)


def kernel(x, edge_attr, edge_index, edge_index2, edge_index3,
           triangle_1_1_1, triangle_1_1_2, triangle_2_2_1, triangle_2_2_2,
           triangle_1_2_3, triangle_3_3_1, triangle_2_2_3, triangle_3_3_2,
           triangle_3_3_3, inverse_edge_1, inverse_edge_2, inverse_edge_3,
           batch0, num_nodes, atom_emb, bond_emb, W1, b1, W2, b2,
           Wp1, bp1, Wp2, bp2):
    nN = x.shape[0]
    nE = edge_attr.shape[0]
    G = 256
    epad = E_PAD - nE

    ei1 = jnp.pad(edge_index, ((0, 0), (0, epad)))
    ei2 = jnp.pad(edge_index2, ((0, 0), (0, epad)))
    ei3 = jnp.pad(edge_index3, ((0, 0), (0, epad)))
    inv1 = jnp.pad(inverse_edge_1, (0, epad))
    inv2 = jnp.pad(inverse_edge_2, (0, epad))
    inv3 = jnp.pad(inverse_edge_3, (0, epad))
    h_atom = _encode(atom_emb, x)
    h_atom = h_atom + jnp.asarray(num_nodes - nN, dtype=h_atom.dtype)
    h0 = h_atom
    e1 = _encode(bond_emb, jnp.pad(edge_attr, ((0, epad), (0, 0))))
    e2 = h_atom[ei2[0]] + h_atom[ei2[1]]
    e3 = h_atom[ei3[0]] + h_atom[ei3[1]]

    segsum_e2n = _build_segsum(((1, nE),), 10240, 10240)
    segsum_node = _build_segsum(((1, nE), (1, nE), (1, nE)), 10240, 10240)
    segsum_pool = _build_segsum(((3, 10240),), 288, 256, counts=True)

    for l in range(W1.shape[0]):
        m0p = segsum_e2n(ei1[1, :nE], e1)
        m1 = (_tri_msg_jnp(e1, e1, triangle_1_1_1, E_PAD)
              + _tri_msg_jnp(e2, e2, triangle_2_2_1, E_PAD)
              + _tri_msg_jnp(e3, e3, triangle_3_3_1, E_PAD)
              + h0[ei1[0]] * h0[ei1[1]])
        m2 = (_tri_msg_jnp(e1, e1, triangle_1_1_2, E_PAD)
              + _tri_msg_jnp(e2, e2, triangle_2_2_2, E_PAD)
              + _tri_msg_jnp(e3, e3, triangle_3_3_2, E_PAD)
              + h0[ei2[0]] * h0[ei2[1]])
        m3 = (_tri_msg_jnp(e1, e2, triangle_1_2_3, E_PAD)
              + _tri_msg_jnp(e2, e2, triangle_2_2_3, E_PAD)
              + _tri_msg_jnp(e3, e3, triangle_3_3_3, E_PAD)
              + h0[ei3[0]] * h0[ei3[1]])
        h0 = _mlp((h0, m0p[0, :nN], m0p[1, :nN]),
                  W1[l, 0], b1[l, 0], W2[l, 0], b2[l, 0],
                  nN, 1000, nN, 1000)
        e1n = _mlp((e1, m1), W1[l, 1], b1[l, 1], W2[l, 1], b2[l, 1],
                   nE, 1280, E_PAD, 1024)
        e1 = 0.5 * (e1n + e1n[inv1])
        e2n = _mlp((e2, m2), W1[l, 2], b1[l, 2], W2[l, 2], b2[l, 2],
                   nE, 1280, E_PAD, 1024)
        e2 = 0.5 * (e2n + e2n[inv2])
        e3n = _mlp((e3, m3), W1[l, 3], b1[l, 3], W2[l, 3], b2[l, 3],
                   nE, 1280, E_PAD, 1024)
        e3 = 0.5 * (e3n + e3n[inv3])

    nsp = segsum_node(ei1[0, :nE], e1, ei2[0, :nE], e2, ei3[0, :nE], e3)
    pad = ((0, 10240 - nN), (0, 0))
    b0p = jnp.pad(batch0, (0, 10240 - nN), constant_values=287)
    gp, cp = segsum_pool(b0p, jnp.pad(h0, pad), jnp.pad(nsp[0, :nN], pad),
                         jnp.pad(nsp[1, :nN], pad))
    cnt = (cp[0] + cp[1])[:, 0:1]
    g = (gp[0] + gp[1]) / jnp.clip(cnt, 1.0)
    return _head(g, Wp1, bp1, Wp2, bp2)


# speedup vs baseline: 17.8598x; 1.0099x over previous
import functools

import jax
import jax.numpy as jnp
from jax import lax
from jax.experimental import pallas as pl
from jax.experimental.pallas import tpu as pltpu
from jax.experimental.pallas import tpu_sc as plsc

D = 128

NC, NS, LN = 2, 16, 16
NW = NC * NS
_SC_MESH = plsc.VectorSubcoreMesh(
    core_axis_name="c", subcore_axis_name="s", num_cores=NC, num_subcores=NS)


def _fill_rows(ref, nrows, value, dtype):
    v = jnp.full((LN,), value, dtype)

    def body(r, _):
        for k in range(D // LN):
            ref[r, pl.ds(LN * k, LN)] = v
        return 0

    lax.fori_loop(0, nrows, body, 0)


def _add_rows(dst, src, nrows):
    def body(r, _):
        for k in range(D // LN):
            dst[r, pl.ds(LN * k, LN)] = (dst[r, pl.ds(LN * k, LN)]
                                         + src[r, pl.ds(LN * k, LN)])
        return 0

    lax.fori_loop(0, nrows, body, 0)


@functools.lru_cache(maxsize=None)
def _build_segsum(phases, S_acc, S_out, counts=False):
    assert S_acc % NS == 0 and S_out % NS == 0
    dummy = S_acc - 1
    max_nblk = 0
    for nv, R in phases:
        share = R // NW
        assert R % NW == 0 and share % 8 == 0
        max_nblk = max(max_nblk, (share + 127) // 128)
    multi = any(nv > 1 for nv, _ in phases)

    out_types = [jax.ShapeDtypeStruct((NC, S_out, D), jnp.float32)]
    if counts:
        out_types.append(jax.ShapeDtypeStruct((NC, S_out, D), jnp.float32))

    scratch = [
        pltpu.VMEM_SHARED((S_acc, D), jnp.float32),
        pltpu.VMEM((max_nblk, 128), jnp.int32),
        pltpu.VMEM((128, D), jnp.float32),
        pltpu.VMEM((128, D), jnp.float32),
        pltpu.SemaphoreType.DMA,
        pltpu.SemaphoreType.DMA,
    ]
    if multi:
        scratch.append(pltpu.VMEM((128, D), jnp.float32))
    if counts:
        scratch.append(pltpu.VMEM_SHARED((S_acc, D), jnp.float32))
        scratch.append(pltpu.VMEM((128, D), jnp.float32))

    def body(*refs):
        n_in = sum(1 + nv for nv, _ in phases)
        ins = refs[:n_in]
        n_out = 2 if counts else 1
        outs = refs[n_in:n_in + n_out]
        scr = list(refs[n_in + n_out:])
        acc = scr.pop(0)
        idx2d = scr.pop(0)
        bufs = [scr.pop(0), scr.pop(0)]
        semI = scr.pop(0)
        semV = scr.pop(0)
        rows2 = scr.pop(0) if multi else None
        cacc = scr.pop(0) if counts else None
        ones = scr.pop(0) if counts else None

        cid = lax.axis_index("c")
        sid = lax.axis_index("s")
        w = cid * NS + sid

        _fill_rows(bufs[0], 128, 0.0, jnp.float32)
        if counts:
            _fill_rows(ones, 128, 1.0, jnp.float32)

        zpt = S_acc // NS
        zdone = 0
        while zdone < zpt:
            zn = min(128, zpt - zdone)
            pltpu.sync_copy(bufs[0].at[pl.ds(0, zn), :],
                            acc.at[pl.ds(sid * zpt + zdone, zn), :])
            if counts:
                pltpu.sync_copy(bufs[0].at[pl.ds(0, zn), :],
                                cacc.at[pl.ds(sid * zpt + zdone, zn), :])
            zdone += zn
        plsc.subcore_barrier()

        argp = 0
        for nv, R in phases:
            idx_hbm = ins[argp]
            vals = ins[argp + 1:argp + 1 + nv]
            argp += 1 + nv
            share = R // NW
            base = w * share
            nb = share // 128
            tail = share % 128
            nblk = nb + (1 if tail else 0)
            _fill_rows(idx2d, nblk, dummy, jnp.int32)
            icps = [pltpu.async_copy(idx_hbm.at[pl.ds(base + 128 * j, 128)],
                                     idx2d.at[j], semI)
                    for j in range(nb)]
            if tail:
                icps.append(pltpu.async_copy(
                    idx_hbm.at[pl.ds(base + 128 * nb, tail)],
                    idx2d.at[nb, pl.ds(0, tail)], semI))
            for cp in icps:
                cp.wait()

            def fire(j):
                ln = 128 if j < nb else tail
                return pltpu.async_copy(
                    vals[0].at[pl.ds(base + 128 * j, ln), :],
                    bufs[j % 2].at[pl.ds(0, ln), :], semV)

            vcp = fire(0)
            for j in range(nblk):
                ln = 128 if j < nb else tail
                vcp.wait()
                if j + 1 < nblk:
                    vcp = fire(j + 1)
                for v in vals[1:]:
                    pltpu.sync_copy(v.at[pl.ds(base + 128 * j, ln), :],
                                    rows2.at[pl.ds(0, ln), :])
                    _add_rows(bufs[j % 2], rows2, ln)
                pltpu.sync_copy(bufs[j % 2], acc.at[idx2d.at[j]], add=True)
                if counts:
                    pltpu.sync_copy(ones, cacc.at[idx2d.at[j]], add=True)
        plsc.subcore_barrier()

        spt = S_out // NS
        pltpu.sync_copy(acc.at[pl.ds(sid * spt, spt), :],
                        outs[0].at[cid, pl.ds(sid * spt, spt), :])
        if counts:
            pltpu.sync_copy(cacc.at[pl.ds(sid * spt, spt), :],
                            outs[1].at[cid, pl.ds(sid * spt, spt), :])

    return pl.kernel(body, out_type=tuple(out_types) if counts else out_types[0],
                     mesh=_SC_MESH, scratch_types=scratch,
                     compiler_params=pltpu.CompilerParams(
                         needs_layout_passes=False))


E_REAL = 160000
C_CH = 4096
NCHUNK = 40
E_PAD = C_CH * NCHUNK
ACC_ROWS = C_CH + 128
T_REAL = 200000
T_PAD = 229376
TSH = T_PAD // NS
TSB = 2048
TSBR = TSB // 128
SELR = TSBR + 2
ESH = C_CH // NS


def _mul_rows(dst, src, nrows):
    def body(r, _):
        for k in range(D // LN):
            dst[r, pl.ds(LN * k, LN)] = (dst[r, pl.ds(LN * k, LN)]
                                         * src[r, pl.ds(LN * k, LN)])
        return 0

    lax.fori_loop(0, nrows, body, 0)


def _trimsg_body(ea0, eb0, t0, ea1, eb1, t1, ea2, eb2, t2, ei0, ei1v, h0,
                 out, acc, s0, s1, s2, g0, g1, d0, ra0, rb0, ra1, rb1,
                 ia, ib, edst, semA, semB):
    cid = lax.axis_index("c")
    sid = lax.axis_index("s")
    dummy = C_CH
    dump = (SELR - 1) * 128
    keys = ((ea0, eb0, t0), (ea1, eb1, t1), (ea2, eb2, t2))

    lane = lax.iota(jnp.int32, LN)
    for j in range(ESH // 128):
        for k in range(128 // LN):
            edst[j, pl.ds(LN * k, LN)] = sid * ESH + 128 * j + LN * k + lane

    def mk_mul(ra, rb):
        def mulblk(r, _):
            for u in range(4):
                for k in range(D // LN):
                    ra[4 * r + u, pl.ds(LN * k, LN)] = (
                        ra[4 * r + u, pl.ds(LN * k, LN)]
                        * rb[4 * r + u, pl.ds(LN * k, LN)])
            return 0

        return mulblk

    mul0 = mk_mul(ra0, rb0)
    mul1 = mk_mul(ra1, rb1)

    def chunk_body(ci, _):
        ch = NC * ci + cid
        lo = ch * C_CH
        hi = lo + C_CH

        _fill_rows(ra0, 128, 0.0, jnp.float32)
        zpt = ACC_ROWS // NS
        zdone = 0
        while zdone < zpt:
            zn = min(128, zpt - zdone)
            pltpu.sync_copy(ra0.at[pl.ds(0, zn), :],
                            acc.at[pl.ds(sid * zpt + zdone, zn), :])
            zdone += zn
        plsc.subcore_barrier()

        for (ea, eb, tr) in keys:
            def block_body(b, _):
                trow = sid * (TSH // 128) + TSBR * b
                cp0 = pltpu.async_copy(tr.at[0, pl.ds(trow, TSBR), :], s0,
                                       semA)
                cp1 = pltpu.async_copy(tr.at[1, pl.ds(trow, TSBR), :], s1,
                                       semA)
                cp2 = pltpu.async_copy(tr.at[2, pl.ds(trow, TSBR), :], s2,
                                       semA)
                cp0.wait()
                cp1.wait()
                cp2.wait()

                def scan_body(i, off):
                    r8, c8 = i // 8, (i % 8) * LN
                    t2v = s2[r8, pl.ds(c8, LN)]
                    m = (t2v >= lo) & (t2v < hi)
                    cum = plsc.cumsum(m.astype(jnp.int32))
                    pos = jnp.where(m, off + cum - 1, dump)
                    pr, pc = pos >> 7, pos & 127
                    plsc.store_scatter(d0, [pr, pc], t2v - lo)
                    plsc.store_scatter(g0, [pr, pc], s0[r8, pl.ds(c8, LN)])
                    plsc.store_scatter(g1, [pr, pc], s1[r8, pl.ds(c8, LN)])
                    return off + cum[LN - 1]

                nsel = lax.fori_loop(0, TSB // LN, scan_body, 0)

                zi = jnp.zeros((LN,), jnp.int32)
                dv = jnp.full((LN,), dummy, jnp.int32)
                for k in range(128 // LN):
                    pos = nsel + LN * k + lane
                    pr, pc = pos >> 7, pos & 127
                    plsc.store_scatter(d0, [pr, pc], dv)
                    plsc.store_scatter(g0, [pr, pc], zi)
                    plsc.store_scatter(g1, [pr, pc], zi)

                nblk = (nsel + 127) // 128

                def fire(j, ra, rb):
                    pltpu.async_copy(ea.at[g0.at[j]], ra, semA)
                    pltpu.async_copy(eb.at[g1.at[j]], rb, semB)

                def drain(ra, rb):
                    pltpu.make_async_copy(ea.at[g0.at[0]], ra, semA).wait()
                    pltpu.make_async_copy(eb.at[g1.at[0]], rb, semB).wait()

                @pl.when(nblk > 0)
                def _():
                    fire(0, ra0, rb0)

                def pair_body(p, _):
                    j0 = 2 * p
                    j1 = j0 + 1

                    @pl.when(j1 < nblk)
                    def _():
                        fire(j1, ra1, rb1)

                    drain(ra0, rb0)
                    lax.fori_loop(0, 32, mul0, 0)
                    pltpu.sync_copy(ra0, acc.at[d0.at[j0]], add=True)

                    @pl.when(j1 < nblk)
                    def _():
                        @pl.when(j1 + 1 < nblk)
                        def _():
                            fire(j1 + 1, ra0, rb0)

                        drain(ra1, rb1)
                        lax.fori_loop(0, 32, mul1, 0)
                        pltpu.sync_copy(ra1, acc.at[d0.at[j1]], add=True)

                    return 0

                lax.fori_loop(0, (nblk + 1) // 2, pair_body, 0)
                return 0

            lax.fori_loop(0, TSH // TSB, block_body, 0)

        ebase = lo + sid * ESH
        for j in range(ESH // 128):
            pltpu.sync_copy(ei0.at[pl.ds(ebase + 128 * j, 128)], ia.at[j])
            pltpu.sync_copy(ei1v.at[pl.ds(ebase + 128 * j, 128)], ib.at[j])
        cpA0 = pltpu.async_copy(h0.at[ia.at[0]], ra0, semA)
        cpB0 = pltpu.async_copy(h0.at[ib.at[0]], rb0, semB)
        cpA1 = pltpu.async_copy(h0.at[ia.at[1]], ra1, semA)
        cpB1 = pltpu.async_copy(h0.at[ib.at[1]], rb1, semB)
        cpA0.wait()
        cpB0.wait()
        lax.fori_loop(0, 32, mul0, 0)
        pltpu.sync_copy(ra0, acc.at[edst.at[0]], add=True)
        cpA1.wait()
        cpB1.wait()
        lax.fori_loop(0, 32, mul1, 0)
        pltpu.sync_copy(ra1, acc.at[edst.at[1]], add=True)
        plsc.subcore_barrier()

        pltpu.sync_copy(acc.at[pl.ds(sid * ESH, ESH), :],
                        out.at[pl.ds(lo + sid * ESH, ESH), :])
        plsc.subcore_barrier()
        return 0

    lax.fori_loop(0, NCHUNK // NC, chunk_body, 0)


@functools.lru_cache(maxsize=None)
def _build_trimsg():
    scratch = [
        pltpu.VMEM_SHARED((ACC_ROWS, D), jnp.float32),
        pltpu.VMEM((TSBR, 128), jnp.int32),
        pltpu.VMEM((TSBR, 128), jnp.int32),
        pltpu.VMEM((TSBR, 128), jnp.int32),
        pltpu.VMEM((SELR, 128), jnp.int32),
        pltpu.VMEM((SELR, 128), jnp.int32),
        pltpu.VMEM((SELR, 128), jnp.int32),
        pltpu.VMEM((128, D), jnp.float32),
        pltpu.VMEM((128, D), jnp.float32),
        pltpu.VMEM((128, D), jnp.float32),
        pltpu.VMEM((128, D), jnp.float32),
        pltpu.VMEM((ESH // 128, 128), jnp.int32),
        pltpu.VMEM((ESH // 128, 128), jnp.int32),
        pltpu.VMEM((ESH // 128, 128), jnp.int32),
        pltpu.SemaphoreType.DMA,
        pltpu.SemaphoreType.DMA,
    ]
    return pl.kernel(
        _trimsg_body,
        out_type=jax.ShapeDtypeStruct((E_PAD, D), jnp.float32),
        mesh=_SC_MESH, scratch_types=scratch,
        compiler_params=pltpu.CompilerParams(needs_layout_passes=False))


def _pad_tri(tri):
    padc = jnp.concatenate(
        [jnp.zeros((2, T_PAD - T_REAL), jnp.int32),
         jnp.full((1, T_PAD - T_REAL), -1, jnp.int32)], axis=0)
    return jnp.concatenate([tri, padc], axis=1).reshape(3, T_PAD // 128, 128)


def _mlp_stats_body(na, *refs):
    addends = refs[:na]
    w1_ref, b1_ref, o_ref = refs[na:]
    i = pl.program_id(0)
    x = addends[0][...]
    for r in addends[1:]:
        x = x + r[...]
    p = jnp.dot(x, w1_ref[...], preferred_element_type=jnp.float32)
    p = p + b1_ref[...]

    @pl.when(i == 0)
    def _():
        o_ref[...] = jnp.zeros_like(o_ref)

    o_ref[0, :] += jnp.sum(p, axis=0)
    o_ref[1, :] += jnp.sum(p * p, axis=0)


def _mlp_apply_body(na, *refs):
    addends = refs[:na]
    w1_ref, b1_ref, st_ref, w2_ref, b2_ref, o_ref = refs[na:]
    x = addends[0][...]
    for r in addends[1:]:
        x = x + r[...]
    p = jnp.dot(x, w1_ref[...], preferred_element_type=jnp.float32)
    p = p + b1_ref[...]
    h = jnp.maximum((p - st_ref[0:1, :]) * st_ref[1:2, :], 0.0)
    o_ref[...] = jnp.dot(h, w2_ref[...], preferred_element_type=jnp.float32) \
        + b2_ref[...]


@functools.partial(jax.jit,
                   static_argnames=("stat_rows", "stat_blk", "out_rows",
                                    "out_blk"))
def _mlp(addends, w1, b1, w2, b2, stat_rows, stat_blk, out_rows, out_blk):
    na = len(addends)
    assert stat_rows % stat_blk == 0 and out_rows % out_blk == 0

    def row_map(i):
        return (i, 0)

    stats = pl.pallas_call(
        functools.partial(_mlp_stats_body, na),
        grid=(stat_rows // stat_blk,),
        in_specs=[pl.BlockSpec((stat_blk, D), row_map)] * na + [
            pl.BlockSpec((D, D), lambda i: (0, 0)),
            pl.BlockSpec((D,), lambda i: (0,)),
        ],
        out_specs=pl.BlockSpec((2, D), lambda i: (0, 0)),
        out_shape=jax.ShapeDtypeStruct((2, D), jnp.float32),
    )(*addends, w1, b1)
    mu = stats[0] * (1.0 / stat_rows)
    var = stats[1] * (1.0 / stat_rows) - mu * mu
    inv_sd = 1.0 / (jnp.sqrt(jnp.maximum(var, 0.0)) + 1e-5)
    st = jnp.stack([mu, inv_sd])

    return pl.pallas_call(
        functools.partial(_mlp_apply_body, na),
        grid=(out_rows // out_blk,),
        in_specs=[pl.BlockSpec((out_blk, D), row_map)] * na + [
            pl.BlockSpec((D, D), lambda i: (0, 0)),
            pl.BlockSpec((D,), lambda i: (0,)),
            pl.BlockSpec((2, D), lambda i: (0, 0)),
            pl.BlockSpec((D, D), lambda i: (0, 0)),
            pl.BlockSpec((D,), lambda i: (0,)),
        ],
        out_specs=pl.BlockSpec((out_blk, D), row_map),
        out_shape=jax.ShapeDtypeStruct((out_rows, D), jnp.float32),
    )(*addends, w1, b1, st, w2, b2)


def _head_body(g_ref, wp1_ref, bp1_ref, wp2_ref, bp2_ref, o_ref):
    h = jnp.dot(g_ref[...], wp1_ref[...], preferred_element_type=jnp.float32)
    h = h + bp1_ref[...]
    h = jnp.where(h > 0, h, jnp.exp(jnp.minimum(h, 0.0)) - 1.0)
    o_ref[...] = jnp.dot(h, wp2_ref[...], preferred_element_type=jnp.float32) \
        + bp2_ref[...]


@jax.jit
def _head(g, wp1, bp1, wp2, bp2):
    return pl.pallas_call(
        _head_body,
        out_shape=jax.ShapeDtypeStruct((g.shape[0], wp2.shape[1]), jnp.float32),
    )(g, wp1, bp1, wp2, bp2)


def _encode(table, idx):
    out = jnp.zeros((idx.shape[0], table.shape[2]), dtype=table.dtype)
    for c in range(table.shape[0]):
        out = out + jnp.take(table[c], idx[:, c], axis=0)
    return out


def _tri_msg_jnp(ea, eb, tri, nseg):
    m = ea[tri[0]] * eb[tri[1]]
    return jax.ops.segment_sum(m, tri[2], num_segments=nseg)


def kernel(x, edge_attr, edge_index, edge_index2, edge_index3,
           triangle_1_1_1, triangle_1_1_2, triangle_2_2_1, triangle_2_2_2,
           triangle_1_2_3, triangle_3_3_1, triangle_2_2_3, triangle_3_3_2,
           triangle_3_3_3, inverse_edge_1, inverse_edge_2, inverse_edge_3,
           batch0, num_nodes, atom_emb, bond_emb, W1, b1, W2, b2,
           Wp1, bp1, Wp2, bp2):
    nN = x.shape[0]
    nE = edge_attr.shape[0]
    G = 256
    epad = E_PAD - nE

    ei1 = jnp.pad(edge_index, ((0, 0), (0, epad)))
    ei2 = jnp.pad(edge_index2, ((0, 0), (0, epad)))
    ei3 = jnp.pad(edge_index3, ((0, 0), (0, epad)))
    inv1 = jnp.pad(inverse_edge_1, (0, epad))
    inv2 = jnp.pad(inverse_edge_2, (0, epad))
    inv3 = jnp.pad(inverse_edge_3, (0, epad))
    h_atom = _encode(atom_emb, x)
    h_atom = h_atom + jnp.asarray(num_nodes - nN, dtype=h_atom.dtype)
    h0 = h_atom
    e1 = _encode(bond_emb, jnp.pad(edge_attr, ((0, epad), (0, 0))))
    e2 = h_atom[ei2[0]] + h_atom[ei2[1]]
    e3 = h_atom[ei3[0]] + h_atom[ei3[1]]

    segsum_e2n = _build_segsum(((1, nE),), 10240, 10240)
    segsum_node = _build_segsum(((1, nE), (1, nE), (1, nE)), 10240, 10240)
    segsum_pool = _build_segsum(((3, 10240),), 288, 256, counts=True)

    for l in range(W1.shape[0]):
        m0p = segsum_e2n(ei1[1, :nE], e1)
        m1 = (_tri_msg_jnp(e1, e1, triangle_1_1_1, E_PAD)
              + _tri_msg_jnp(e2, e2, triangle_2_2_1, E_PAD)
              + _tri_msg_jnp(e3, e3, triangle_3_3_1, E_PAD)
              + h0[ei1[0]] * h0[ei1[1]])
        m2 = (_tri_msg_jnp(e1, e1, triangle_1_1_2, E_PAD)
              + _tri_msg_jnp(e2, e2, triangle_2_2_2, E_PAD)
              + _tri_msg_jnp(e3, e3, triangle_3_3_2, E_PAD)
              + h0[ei2[0]] * h0[ei2[1]])
        m3 = (_tri_msg_jnp(e1, e2, triangle_1_2_3, E_PAD)
              + _tri_msg_jnp(e2, e2, triangle_2_2_3, E_PAD)
              + _tri_msg_jnp(e3, e3, triangle_3_3_3, E_PAD)
              + h0[ei3[0]] * h0[ei3[1]])
        h0 = _mlp((h0, m0p[0, :nN], m0p[1, :nN]),
                  W1[l, 0], b1[l, 0], W2[l, 0], b2[l, 0],
                  nN, 1000, nN, 1000)
        e1n = _mlp((e1, m1), W1[l, 1], b1[l, 1], W2[l, 1], b2[l, 1],
                   nE, 1280, E_PAD, 1024)
        e1 = 0.5 * (e1n + e1n[inv1])
        e2n = _mlp((e2, m2), W1[l, 2], b1[l, 2], W2[l, 2], b2[l, 2],
                   nE, 1280, E_PAD, 1024)
        e2 = 0.5 * (e2n + e2n[inv2])
        e3n = _mlp((e3, m3), W1[l, 3], b1[l, 3], W2[l, 3], b2[l, 3],
                   nE, 1280, E_PAD, 1024)
        e3 = 0.5 * (e3n + e3n[inv3])

    nsp = segsum_node(ei1[0, :nE], e1, ei2[0, :nE], e2, ei3[0, :nE], e3)
    pad = ((0, 10240 - nN), (0, 0))
    b0p = jnp.pad(batch0, (0, 10240 - nN), constant_values=287)
    gp, cp = segsum_pool(b0p, jnp.pad(h0, pad), jnp.pad(nsp[0, :nN], pad),
                         jnp.pad(nsp[1, :nN], pad))
    cnt = (cp[0] + cp[1])[:, 0:1]
    g = (gp[0] + gp[1]) / jnp.clip(cnt, 1.0)
    return _head(g, Wp1, bp1, Wp2, bp2)


# speedup vs baseline: 19.8523x; 1.1116x over previous
import functools

import jax
import jax.numpy as jnp
from jax import lax
from jax.experimental import pallas as pl
from jax.experimental.pallas import tpu as pltpu
from jax.experimental.pallas import tpu_sc as plsc

D = 128

NC, NS, LN = 2, 16, 16
NW = NC * NS
_SC_MESH = plsc.VectorSubcoreMesh(
    core_axis_name="c", subcore_axis_name="s", num_cores=NC, num_subcores=NS)


def _fill_rows(ref, nrows, value, dtype):
    v = jnp.full((LN,), value, dtype)

    def body(r, _):
        for k in range(D // LN):
            ref[r, pl.ds(LN * k, LN)] = v
        return 0

    lax.fori_loop(0, nrows, body, 0)


def _add_rows(dst, src, nrows):
    def body(r, _):
        for k in range(D // LN):
            dst[r, pl.ds(LN * k, LN)] = (dst[r, pl.ds(LN * k, LN)]
                                         + src[r, pl.ds(LN * k, LN)])
        return 0

    lax.fori_loop(0, nrows, body, 0)


@functools.lru_cache(maxsize=None)
def _build_segsum(phases, S_acc, S_out, counts=False):
    assert S_acc % NS == 0 and S_out % NS == 0
    dummy = S_acc - 1
    max_nblk = 0
    for nv, R in phases:
        share = R // NW
        assert R % NW == 0 and share % 8 == 0
        max_nblk = max(max_nblk, (share + 127) // 128)
    multi = any(nv > 1 for nv, _ in phases)

    out_types = [jax.ShapeDtypeStruct((NC, S_out, D), jnp.float32)]
    if counts:
        out_types.append(jax.ShapeDtypeStruct((NC, S_out, D), jnp.float32))

    scratch = [
        pltpu.VMEM_SHARED((S_acc, D), jnp.float32),
        pltpu.VMEM((max_nblk, 128), jnp.int32),
        pltpu.VMEM((128, D), jnp.float32),
        pltpu.VMEM((128, D), jnp.float32),
        pltpu.SemaphoreType.DMA,
        pltpu.SemaphoreType.DMA,
    ]
    if multi:
        scratch.append(pltpu.VMEM((128, D), jnp.float32))
    if counts:
        scratch.append(pltpu.VMEM_SHARED((S_acc, D), jnp.float32))
        scratch.append(pltpu.VMEM((128, D), jnp.float32))

    def body(*refs):
        n_in = sum(1 + nv for nv, _ in phases)
        ins = refs[:n_in]
        n_out = 2 if counts else 1
        outs = refs[n_in:n_in + n_out]
        scr = list(refs[n_in + n_out:])
        acc = scr.pop(0)
        idx2d = scr.pop(0)
        bufs = [scr.pop(0), scr.pop(0)]
        semI = scr.pop(0)
        semV = scr.pop(0)
        rows2 = scr.pop(0) if multi else None
        cacc = scr.pop(0) if counts else None
        ones = scr.pop(0) if counts else None

        cid = lax.axis_index("c")
        sid = lax.axis_index("s")
        w = cid * NS + sid

        _fill_rows(bufs[0], 128, 0.0, jnp.float32)
        if counts:
            _fill_rows(ones, 128, 1.0, jnp.float32)

        zpt = S_acc // NS
        zdone = 0
        while zdone < zpt:
            zn = min(128, zpt - zdone)
            pltpu.sync_copy(bufs[0].at[pl.ds(0, zn), :],
                            acc.at[pl.ds(sid * zpt + zdone, zn), :])
            if counts:
                pltpu.sync_copy(bufs[0].at[pl.ds(0, zn), :],
                                cacc.at[pl.ds(sid * zpt + zdone, zn), :])
            zdone += zn
        plsc.subcore_barrier()

        argp = 0
        for nv, R in phases:
            idx_hbm = ins[argp]
            vals = ins[argp + 1:argp + 1 + nv]
            argp += 1 + nv
            share = R // NW
            base = w * share
            nb = share // 128
            tail = share % 128
            nblk = nb + (1 if tail else 0)
            _fill_rows(idx2d, nblk, dummy, jnp.int32)
            icps = [pltpu.async_copy(idx_hbm.at[pl.ds(base + 128 * j, 128)],
                                     idx2d.at[j], semI)
                    for j in range(nb)]
            if tail:
                icps.append(pltpu.async_copy(
                    idx_hbm.at[pl.ds(base + 128 * nb, tail)],
                    idx2d.at[nb, pl.ds(0, tail)], semI))
            for cp in icps:
                cp.wait()

            def fire(j):
                ln = 128 if j < nb else tail
                return pltpu.async_copy(
                    vals[0].at[pl.ds(base + 128 * j, ln), :],
                    bufs[j % 2].at[pl.ds(0, ln), :], semV)

            vcp = fire(0)
            for j in range(nblk):
                ln = 128 if j < nb else tail
                vcp.wait()
                if j + 1 < nblk:
                    vcp = fire(j + 1)
                for v in vals[1:]:
                    pltpu.sync_copy(v.at[pl.ds(base + 128 * j, ln), :],
                                    rows2.at[pl.ds(0, ln), :])
                    _add_rows(bufs[j % 2], rows2, ln)
                pltpu.sync_copy(bufs[j % 2], acc.at[idx2d.at[j]], add=True)
                if counts:
                    pltpu.sync_copy(ones, cacc.at[idx2d.at[j]], add=True)
        plsc.subcore_barrier()

        spt = S_out // NS
        pltpu.sync_copy(acc.at[pl.ds(sid * spt, spt), :],
                        outs[0].at[cid, pl.ds(sid * spt, spt), :])
        if counts:
            pltpu.sync_copy(cacc.at[pl.ds(sid * spt, spt), :],
                            outs[1].at[cid, pl.ds(sid * spt, spt), :])

    return pl.kernel(body, out_type=tuple(out_types) if counts else out_types[0],
                     mesh=_SC_MESH, scratch_types=scratch,
                     compiler_params=pltpu.CompilerParams(
                         needs_layout_passes=False))


E_REAL = 160000
C_CH = 4096
NCHUNK = 40
E_PAD = C_CH * NCHUNK
ACC_ROWS = C_CH + 128
T_REAL = 200000
T_PAD = 229376
TSH = T_PAD // NS
TSB = 2048
TSBR = TSB // 128
SELR = TSBR + 2
ESH = C_CH // NS


def _mul_rows(dst, src, nrows):
    def body(r, _):
        for k in range(D // LN):
            dst[r, pl.ds(LN * k, LN)] = (dst[r, pl.ds(LN * k, LN)]
                                         * src[r, pl.ds(LN * k, LN)])
        return 0

    lax.fori_loop(0, nrows, body, 0)


def _trimsg_body(ea0, eb0, t0, ea1, eb1, t1, ea2, eb2, t2, ei0, ei1v, h0,
                 out, acc, s0, s1, s2, g0, g1, d0, ra0, rb0, ra1, rb1,
                 ia, ib, edst, semA, semB):
    cid = lax.axis_index("c")
    sid = lax.axis_index("s")
    dummy = C_CH
    dump = (SELR - 1) * 128
    keys = ((ea0, eb0, t0), (ea1, eb1, t1), (ea2, eb2, t2))

    lane = lax.iota(jnp.int32, LN)
    for j in range(ESH // 128):
        for k in range(128 // LN):
            edst[j, pl.ds(LN * k, LN)] = sid * ESH + 128 * j + LN * k + lane

    def mk_mul(ra, rb):
        def mulblk(r, _):
            for u in range(4):
                for k in range(D // LN):
                    ra[4 * r + u, pl.ds(LN * k, LN)] = (
                        ra[4 * r + u, pl.ds(LN * k, LN)]
                        * rb[4 * r + u, pl.ds(LN * k, LN)])
            return 0

        return mulblk

    mul0 = mk_mul(ra0, rb0)
    mul1 = mk_mul(ra1, rb1)

    def chunk_body(ci, _):
        ch = NC * ci + cid
        lo = ch * C_CH
        hi = lo + C_CH

        _fill_rows(ra0, 128, 0.0, jnp.float32)
        zpt = ACC_ROWS // NS
        zdone = 0
        while zdone < zpt:
            zn = min(128, zpt - zdone)
            pltpu.sync_copy(ra0.at[pl.ds(0, zn), :],
                            acc.at[pl.ds(sid * zpt + zdone, zn), :])
            zdone += zn
        plsc.subcore_barrier()

        for (ea, eb, tr) in keys:
            def block_body(b, _):
                trow = sid * (TSH // 128) + TSBR * b
                cp0 = pltpu.async_copy(tr.at[0, pl.ds(trow, TSBR), :], s0,
                                       semA)
                cp1 = pltpu.async_copy(tr.at[1, pl.ds(trow, TSBR), :], s1,
                                       semA)
                cp2 = pltpu.async_copy(tr.at[2, pl.ds(trow, TSBR), :], s2,
                                       semA)
                cp0.wait()
                cp1.wait()
                cp2.wait()

                def scan_body(i, off):
                    r8, c8 = i // 8, (i % 8) * LN
                    t2v = s2[r8, pl.ds(c8, LN)]
                    m = (t2v >= lo) & (t2v < hi)
                    cum = plsc.cumsum(m.astype(jnp.int32))
                    pos = jnp.where(m, off + cum - 1, dump)
                    pr, pc = pos >> 7, pos & 127
                    plsc.store_scatter(d0, [pr, pc], t2v - lo)
                    plsc.store_scatter(g0, [pr, pc], s0[r8, pl.ds(c8, LN)])
                    plsc.store_scatter(g1, [pr, pc], s1[r8, pl.ds(c8, LN)])
                    return off + cum[LN - 1]

                nsel = lax.fori_loop(0, TSB // LN, scan_body, 0)

                zi = jnp.zeros((LN,), jnp.int32)
                dv = jnp.full((LN,), dummy, jnp.int32)
                for k in range(128 // LN):
                    pos = nsel + LN * k + lane
                    pr, pc = pos >> 7, pos & 127
                    plsc.store_scatter(d0, [pr, pc], dv)
                    plsc.store_scatter(g0, [pr, pc], zi)
                    plsc.store_scatter(g1, [pr, pc], zi)

                nblk = (nsel + 127) // 128

                def fire(j, ra, rb):
                    pltpu.async_copy(ea.at[g0.at[j]], ra, semA)
                    pltpu.async_copy(eb.at[g1.at[j]], rb, semB)

                def drain(ra, rb):
                    pltpu.make_async_copy(ea.at[g0.at[0]], ra, semA).wait()
                    pltpu.make_async_copy(eb.at[g1.at[0]], rb, semB).wait()

                @pl.when(nblk > 0)
                def _():
                    fire(0, ra0, rb0)

                def pair_body(p, _):
                    j0 = 2 * p
                    j1 = j0 + 1

                    @pl.when(j1 < nblk)
                    def _():
                        fire(j1, ra1, rb1)

                    drain(ra0, rb0)
                    lax.fori_loop(0, 32, mul0, 0)
                    pltpu.sync_copy(ra0, acc.at[d0.at[j0]], add=True)

                    @pl.when(j1 < nblk)
                    def _():
                        @pl.when(j1 + 1 < nblk)
                        def _():
                            fire(j1 + 1, ra0, rb0)

                        drain(ra1, rb1)
                        lax.fori_loop(0, 32, mul1, 0)
                        pltpu.sync_copy(ra1, acc.at[d0.at[j1]], add=True)

                    return 0

                lax.fori_loop(0, (nblk + 1) // 2, pair_body, 0)
                return 0

            lax.fori_loop(0, TSH // TSB, block_body, 0)

        ebase = lo + sid * ESH
        for j in range(ESH // 128):
            pltpu.sync_copy(ei0.at[pl.ds(ebase + 128 * j, 128)], ia.at[j])
            pltpu.sync_copy(ei1v.at[pl.ds(ebase + 128 * j, 128)], ib.at[j])
        cpA0 = pltpu.async_copy(h0.at[ia.at[0]], ra0, semA)
        cpB0 = pltpu.async_copy(h0.at[ib.at[0]], rb0, semB)
        cpA1 = pltpu.async_copy(h0.at[ia.at[1]], ra1, semA)
        cpB1 = pltpu.async_copy(h0.at[ib.at[1]], rb1, semB)
        cpA0.wait()
        cpB0.wait()
        lax.fori_loop(0, 32, mul0, 0)
        pltpu.sync_copy(ra0, acc.at[edst.at[0]], add=True)
        cpA1.wait()
        cpB1.wait()
        lax.fori_loop(0, 32, mul1, 0)
        pltpu.sync_copy(ra1, acc.at[edst.at[1]], add=True)
        plsc.subcore_barrier()

        pltpu.sync_copy(acc.at[pl.ds(sid * ESH, ESH), :],
                        out.at[pl.ds(lo + sid * ESH, ESH), :])
        plsc.subcore_barrier()
        return 0

    lax.fori_loop(0, NCHUNK // NC, chunk_body, 0)


@functools.lru_cache(maxsize=None)
def _build_trimsg():
    scratch = [
        pltpu.VMEM_SHARED((ACC_ROWS, D), jnp.float32),
        pltpu.VMEM((TSBR, 128), jnp.int32),
        pltpu.VMEM((TSBR, 128), jnp.int32),
        pltpu.VMEM((TSBR, 128), jnp.int32),
        pltpu.VMEM((SELR, 128), jnp.int32),
        pltpu.VMEM((SELR, 128), jnp.int32),
        pltpu.VMEM((SELR, 128), jnp.int32),
        pltpu.VMEM((128, D), jnp.float32),
        pltpu.VMEM((128, D), jnp.float32),
        pltpu.VMEM((128, D), jnp.float32),
        pltpu.VMEM((128, D), jnp.float32),
        pltpu.VMEM((ESH // 128, 128), jnp.int32),
        pltpu.VMEM((ESH // 128, 128), jnp.int32),
        pltpu.VMEM((ESH // 128, 128), jnp.int32),
        pltpu.SemaphoreType.DMA,
        pltpu.SemaphoreType.DMA,
    ]
    return pl.kernel(
        _trimsg_body,
        out_type=jax.ShapeDtypeStruct((E_PAD, D), jnp.float32),
        mesh=_SC_MESH, scratch_types=scratch,
        compiler_params=pltpu.CompilerParams(needs_layout_passes=False))


def _pad_tri(tri):
    padc = jnp.concatenate(
        [jnp.zeros((2, T_PAD - T_REAL), jnp.int32),
         jnp.full((1, T_PAD - T_REAL), -1, jnp.int32)], axis=0)
    return jnp.concatenate([tri, padc], axis=1).reshape(3, T_PAD // 128, 128)


def _mlp_stats_body(na, *refs):
    addends = refs[:na]
    w1_ref, b1_ref, o_ref = refs[na:]
    i = pl.program_id(0)
    x = addends[0][...]
    for r in addends[1:]:
        x = x + r[...]
    p = jnp.dot(x, w1_ref[...], preferred_element_type=jnp.float32)
    p = p + b1_ref[...]

    @pl.when(i == 0)
    def _():
        o_ref[...] = jnp.zeros_like(o_ref)

    o_ref[0, :] += jnp.sum(p, axis=0)
    o_ref[1, :] += jnp.sum(p * p, axis=0)


def _mlp_apply_body(na, *refs):
    addends = refs[:na]
    w1_ref, b1_ref, st_ref, w2_ref, b2_ref, o_ref = refs[na:]
    x = addends[0][...]
    for r in addends[1:]:
        x = x + r[...]
    p = jnp.dot(x, w1_ref[...], preferred_element_type=jnp.float32)
    p = p + b1_ref[...]
    h = jnp.maximum((p - st_ref[0:1, :]) * st_ref[1:2, :], 0.0)
    o_ref[...] = jnp.dot(h, w2_ref[...], preferred_element_type=jnp.float32) \
        + b2_ref[...]


@functools.partial(jax.jit,
                   static_argnames=("stat_rows", "stat_blk", "out_rows",
                                    "out_blk"))
def _mlp(addends, w1, b1, w2, b2, stat_rows, stat_blk, out_rows, out_blk):
    na = len(addends)
    assert stat_rows % stat_blk == 0 and out_rows % out_blk == 0

    def row_map(i):
        return (i, 0)

    stats = pl.pallas_call(
        functools.partial(_mlp_stats_body, na),
        grid=(stat_rows // stat_blk,),
        in_specs=[pl.BlockSpec((stat_blk, D), row_map)] * na + [
            pl.BlockSpec((D, D), lambda i: (0, 0)),
            pl.BlockSpec((D,), lambda i: (0,)),
        ],
        out_specs=pl.BlockSpec((2, D), lambda i: (0, 0)),
        out_shape=jax.ShapeDtypeStruct((2, D), jnp.float32),
    )(*addends, w1, b1)
    mu = stats[0] * (1.0 / stat_rows)
    var = stats[1] * (1.0 / stat_rows) - mu * mu
    inv_sd = 1.0 / (jnp.sqrt(jnp.maximum(var, 0.0)) + 1e-5)
    st = jnp.stack([mu, inv_sd])

    return pl.pallas_call(
        functools.partial(_mlp_apply_body, na),
        grid=(out_rows // out_blk,),
        in_specs=[pl.BlockSpec((out_blk, D), row_map)] * na + [
            pl.BlockSpec((D, D), lambda i: (0, 0)),
            pl.BlockSpec((D,), lambda i: (0,)),
            pl.BlockSpec((2, D), lambda i: (0, 0)),
            pl.BlockSpec((D, D), lambda i: (0, 0)),
            pl.BlockSpec((D,), lambda i: (0,)),
        ],
        out_specs=pl.BlockSpec((out_blk, D), row_map),
        out_shape=jax.ShapeDtypeStruct((out_rows, D), jnp.float32),
    )(*addends, w1, b1, st, w2, b2)


def _head_body(g_ref, wp1_ref, bp1_ref, wp2_ref, bp2_ref, o_ref):
    h = jnp.dot(g_ref[...], wp1_ref[...], preferred_element_type=jnp.float32)
    h = h + bp1_ref[...]
    h = jnp.where(h > 0, h, jnp.exp(jnp.minimum(h, 0.0)) - 1.0)
    o_ref[...] = jnp.dot(h, wp2_ref[...], preferred_element_type=jnp.float32) \
        + bp2_ref[...]


@jax.jit
def _head(g, wp1, bp1, wp2, bp2):
    return pl.pallas_call(
        _head_body,
        out_shape=jax.ShapeDtypeStruct((g.shape[0], wp2.shape[1]), jnp.float32),
    )(g, wp1, bp1, wp2, bp2)


def _encode(table, idx):
    out = jnp.zeros((idx.shape[0], table.shape[2]), dtype=table.dtype)
    for c in range(table.shape[0]):
        out = out + jnp.take(table[c], idx[:, c], axis=0)
    return out


def _tri_msg_jnp(ea, eb, tri, nseg):
    m = ea[tri[0]] * eb[tri[1]]
    return jax.ops.segment_sum(m, tri[2], num_segments=nseg)


def kernel(x, edge_attr, edge_index, edge_index2, edge_index3,
           triangle_1_1_1, triangle_1_1_2, triangle_2_2_1, triangle_2_2_2,
           triangle_1_2_3, triangle_3_3_1, triangle_2_2_3, triangle_3_3_2,
           triangle_3_3_3, inverse_edge_1, inverse_edge_2, inverse_edge_3,
           batch0, num_nodes, atom_emb, bond_emb, W1, b1, W2, b2,
           Wp1, bp1, Wp2, bp2):
    nN = x.shape[0]
    nE = edge_attr.shape[0]
    G = 256
    ei1, ei2, ei3 = edge_index, edge_index2, edge_index3
    inv1, inv2, inv3 = inverse_edge_1, inverse_edge_2, inverse_edge_3
    h_atom = _encode(atom_emb, x)
    h_atom = h_atom + jnp.asarray(num_nodes - nN, dtype=h_atom.dtype)
    h0 = h_atom
    e1 = _encode(bond_emb, edge_attr)
    e2 = h_atom[ei2[0]] + h_atom[ei2[1]]
    e3 = h_atom[ei3[0]] + h_atom[ei3[1]]

    segsum_e2n = _build_segsum(((1, nE),), 10240, 10240)
    segsum_node = _build_segsum(((1, nE), (1, nE), (1, nE)), 10240, 10240)
    segsum_pool = _build_segsum(((3, 10240),), 288, 256, counts=True)

    for l in range(W1.shape[0]):
        m0p = segsum_e2n(ei1[1], e1)
        m1 = (_tri_msg_jnp(e1, e1, triangle_1_1_1, nE)
              + _tri_msg_jnp(e2, e2, triangle_2_2_1, nE)
              + _tri_msg_jnp(e3, e3, triangle_3_3_1, nE)
              + h0[ei1[0]] * h0[ei1[1]])
        m2 = (_tri_msg_jnp(e1, e1, triangle_1_1_2, nE)
              + _tri_msg_jnp(e2, e2, triangle_2_2_2, nE)
              + _tri_msg_jnp(e3, e3, triangle_3_3_2, nE)
              + h0[ei2[0]] * h0[ei2[1]])
        m3 = (_tri_msg_jnp(e1, e2, triangle_1_2_3, nE)
              + _tri_msg_jnp(e2, e2, triangle_2_2_3, nE)
              + _tri_msg_jnp(e3, e3, triangle_3_3_3, nE)
              + h0[ei3[0]] * h0[ei3[1]])
        h0 = _mlp((h0, m0p[0, :nN], m0p[1, :nN]),
                  W1[l, 0], b1[l, 0], W2[l, 0], b2[l, 0],
                  nN, 1000, nN, 1000)
        e1n = _mlp((e1, m1), W1[l, 1], b1[l, 1], W2[l, 1], b2[l, 1],
                   nE, 1280, nE, 1280)
        e1 = 0.5 * (e1n + e1n[inv1])
        e2n = _mlp((e2, m2), W1[l, 2], b1[l, 2], W2[l, 2], b2[l, 2],
                   nE, 1280, nE, 1280)
        e2 = 0.5 * (e2n + e2n[inv2])
        e3n = _mlp((e3, m3), W1[l, 3], b1[l, 3], W2[l, 3], b2[l, 3],
                   nE, 1280, nE, 1280)
        e3 = 0.5 * (e3n + e3n[inv3])

    nsp = segsum_node(ei1[0], e1, ei2[0], e2, ei3[0], e3)
    pad = ((0, 10240 - nN), (0, 0))
    b0p = jnp.pad(batch0, (0, 10240 - nN), constant_values=287)
    gp, cp = segsum_pool(b0p, jnp.pad(h0, pad), jnp.pad(nsp[0, :nN], pad),
                         jnp.pad(nsp[1, :nN], pad))
    cnt = (cp[0] + cp[1])[:, 0:1]
    g = (gp[0] + gp[1]) / jnp.clip(cnt, 1.0)
    return _head(g, Wp1, bp1, Wp2, bp2)
